# baseline (speedup 1.0000x reference)
import functools

import jax
import jax.numpy as jnp
from jax import lax
from jax.experimental import pallas as pl
from jax.experimental.pallas import tpu as pltpu
from jax.experimental.pallas import tpu_sc as plsc

_L = 16
_IDX_CHUNK = 128


@functools.lru_cache(maxsize=None)
def _build(cap, push_b, n, d):
    info = plsc.get_sparse_core_info()
    nw = info.num_cores * info.num_subcores
    bpw = n // nw

    mesh = plsc.VectorSubcoreMesh(core_axis_name="c", subcore_axis_name="s")

    @functools.partial(
        pl.kernel,
        mesh=mesh,
        out_type=jax.ShapeDtypeStruct((n, d), jnp.float32),
        compiler_params=pltpu.CompilerParams(use_tc_tiling_on_sc=False),
        scratch_types=[
            pltpu.VMEM((bpw,), jnp.int32),
            pltpu.VMEM((bpw,), jnp.int32),
            pltpu.VMEM((bpw,), jnp.int32),
            pltpu.VMEM((_L,), jnp.int32),
            pltpu.VMEM((bpw, d), jnp.float32),
            pltpu.VMEM((bpw, d), jnp.float32),
            pltpu.SemaphoreType.DMA,
        ],
    )
    def sc_kernel(buf_hbm, val_hbm, ptr_hbm, sidx_hbm, out_hbm,
                  idx_v, vidx_v, wm_v, ptr_v, rows_b, rows_v, sem):
        wid = lax.axis_index("s") * info.num_cores + lax.axis_index("c")
        base = wid * bpw

        pltpu.sync_copy(sidx_hbm.at[pl.ds(base, bpw)], idx_v)
        pltpu.sync_copy(ptr_hbm, ptr_v)
        ptrv = ptr_v[...]

        zero = jnp.zeros((_L,), jnp.int32)
        one = jnp.ones((_L,), jnp.int32)
        capv = jnp.full((_L,), cap, jnp.int32)
        pbv = jnp.full((_L,), push_b, jnp.int32)

        for t in range(bpw // _L):
            s = idx_v[pl.ds(t * _L, _L)]
            off = s - ptrv
            off = jnp.where(off < zero, off + capv, off)
            w = off < pbv
            vidx_v[pl.ds(t * _L, _L)] = jnp.where(w, off, zero)
            wm_v[pl.ds(t * _L, _L)] = jnp.where(w, one, zero)

        copies = []
        for h in range(bpw // _IDX_CHUNK):
            sl = pl.ds(h * _IDX_CHUNK, _IDX_CHUNK)
            copies.append(pltpu.async_copy(
                buf_hbm.at[idx_v.at[sl]], rows_b.at[sl], sem))
            copies.append(pltpu.async_copy(
                val_hbm.at[vidx_v.at[sl]], rows_v.at[sl], sem))
        for cp in copies:
            cp.wait()

        def body(g, carry):
            base_row = g * _L
            wv = wm_v[pl.ds(base_row, _L)]
            for k in range(_L):
                m = wv[k]

                @pl.when(m != 0)
                def _(k=k):
                    j = base_row + k
                    for c in range(d // _L):
                        sl = pl.ds(c * _L, _L)
                        rows_b[j, sl] = rows_v[j, sl]

            return carry

        lax.fori_loop(0, bpw // _L, body, 0)

        pltpu.sync_copy(rows_b, out_hbm.at[pl.ds(base, bpw)])

    return sc_kernel


def kernel(buffer, val, ptr, sample_idx):
    cap, d = buffer.shape
    push_b = val.shape[0]
    n = sample_idx.shape[0]
    ptr_vec = jnp.full((_L,), ptr, dtype=jnp.int32)
    sc = _build(cap, push_b, n, d)
    return sc(buffer, val, ptr_vec, sample_idx.astype(jnp.int32))
